# baseline (speedup 1.0000x reference)
import functools

import jax
import jax.numpy as jnp
from jax import lax
from jax.experimental import pallas as pl
from jax.experimental.pallas import tpu as pltpu
from jax.experimental.pallas import tpu_sc as plsc

_LANES = 16


def kernel(x, pos_embedding):
    B, S, D = x.shape
    info = plsc.get_sparse_core_info()
    NC, NS = info.num_cores, info.num_subcores
    NW = NC * NS
    SPW = S // NW
    CH = 16
    NCH = SPW // CH

    mesh = plsc.VectorSubcoreMesh(core_axis_name="c", subcore_axis_name="s")

    @functools.partial(
        pl.kernel,
        out_type=jax.ShapeDtypeStruct((B, S, D), jnp.float32),
        mesh=mesh,
        scratch_types=[
            pltpu.VMEM((CH, D), jnp.float32),
            pltpu.VMEM((CH, D), jnp.float32),
        ],
    )
    def sc_add(x_hbm, pos_hbm, out_hbm, xbuf, posbuf):
        wid = lax.axis_index("s") * NC + lax.axis_index("c")
        base = wid * SPW

        def chunk_body(c, carry):
            sbase = base + c * CH
            pltpu.sync_copy(pos_hbm.at[pl.ds(sbase, CH)], posbuf)
            for b in range(B):
                pltpu.sync_copy(x_hbm.at[b, pl.ds(sbase, CH)], xbuf)

                def row_body(r, _):
                    def vec_body(j, _):
                        sl = pl.ds(j * _LANES, _LANES)
                        xbuf[r, sl] = xbuf[r, sl] + posbuf[r, sl]
                        return 0

                    return lax.fori_loop(0, D // _LANES, vec_body, 0)

                lax.fori_loop(0, CH, row_body, 0)
                pltpu.sync_copy(xbuf, out_hbm.at[b, pl.ds(sbase, CH)])
            return carry

        lax.fori_loop(0, NCH, chunk_body, 0)

    return sc_add(x, pos_embedding)


# speedup vs baseline: 2.0204x; 2.0204x over previous
import functools

import jax
import jax.numpy as jnp
from jax import lax
from jax.experimental import pallas as pl
from jax.experimental.pallas import tpu as pltpu
from jax.experimental.pallas import tpu_sc as plsc

_LANES = 16


def _tc_add_kernel(x_ref, p_ref, o_ref):
    o_ref[...] = x_ref[...] + p_ref[...]


def _tc_part(xr, pos_embedding, n_rows, S, D, SBLK):
    grid = n_rows // SBLK
    nps = S // SBLK
    return pl.pallas_call(
        _tc_add_kernel,
        grid=(grid,),
        in_specs=[
            pl.BlockSpec((SBLK, D), lambda i: (i, 0)),
            pl.BlockSpec((SBLK, D), lambda i, nps=nps: (i % nps, 0)),
        ],
        out_specs=pl.BlockSpec((SBLK, D), lambda i: (i, 0)),
        out_shape=jax.ShapeDtypeStruct((n_rows, D), xr.dtype),
    )(xr, pos_embedding)


def _sc_part(xr, pos_embedding, row0, n_rows, S, D):
    info = plsc.get_sparse_core_info()
    NW = info.num_cores * info.num_subcores
    SPW = n_rows // NW
    CH = 16
    NCH = SPW // CH
    mesh = plsc.VectorSubcoreMesh(core_axis_name="c", subcore_axis_name="s")

    @functools.partial(
        pl.kernel,
        out_type=jax.ShapeDtypeStruct((n_rows, D), jnp.float32),
        mesh=mesh,
        scratch_types=[
            pltpu.VMEM((CH, D), jnp.float32),
            pltpu.VMEM((CH, D), jnp.float32),
        ],
    )
    def sc_add(x_hbm, pos_hbm, out_hbm, xbuf, posbuf):
        wid = lax.axis_index("s") * info.num_cores + lax.axis_index("c")
        base = wid * SPW

        def chunk_body(c, carry):
            rbase = base + c * CH
            sbase = (row0 + rbase) % S
            pltpu.sync_copy(pos_hbm.at[pl.ds(sbase, CH)], posbuf)
            pltpu.sync_copy(x_hbm.at[pl.ds(row0 + rbase, CH)], xbuf)

            def row_body(r, _):
                def vec_body(j, _):
                    sl = pl.ds(j * _LANES, _LANES)
                    xbuf[r, sl] = xbuf[r, sl] + posbuf[r, sl]
                    return 0

                return lax.fori_loop(0, D // _LANES, vec_body, 0)

            lax.fori_loop(0, CH, row_body, 0)
            pltpu.sync_copy(xbuf, out_hbm.at[pl.ds(rbase, CH)])
            return carry

        lax.fori_loop(0, NCH, chunk_body, 0)

    return sc_add(xr, pos_embedding)


def kernel(x, pos_embedding):
    B, S, D = x.shape
    R = B * S
    xr = x.reshape(R, D)
    n_sc = R // 8
    n_tc = R - n_sc
    tc_out = _tc_part(xr, pos_embedding, n_tc, S, D, SBLK=512)
    sc_out = _sc_part(xr, pos_embedding, n_tc, n_sc, S, D)
    return jnp.concatenate([tc_out, sc_out], axis=0).reshape(B, S, D)


# speedup vs baseline: 5.2541x; 2.6005x over previous
import jax
import jax.numpy as jnp
from jax.experimental import pallas as pl


def _add_kernel(x_ref, p_ref, o_ref):
    o_ref[...] = x_ref[...] + p_ref[...][None, :, :]


def kernel(x, pos_embedding):
    B, S, D = x.shape
    SBLK = 512
    return pl.pallas_call(
        _add_kernel,
        grid=(S // SBLK,),
        in_specs=[
            pl.BlockSpec((B, SBLK, D), lambda s: (0, s, 0)),
            pl.BlockSpec((SBLK, D), lambda s: (s, 0)),
        ],
        out_specs=pl.BlockSpec((B, SBLK, D), lambda s: (0, s, 0)),
        out_shape=jax.ShapeDtypeStruct((B, S, D), x.dtype),
    )(x, pos_embedding)
